# Initial kernel scaffold; baseline (speedup 1.0000x reference)
import jax
import jax.numpy as jnp
from jax.experimental import pallas as pl


def kernel(x, Wx, Wh, b):
    raise NotImplementedError("write your pallas kernel here")


# speedup vs baseline: 2.6307x; 2.6307x over previous
import jax
import jax.numpy as jnp
from jax.experimental import pallas as pl
from jax.experimental.pallas import tpu as pltpu

HID = 32
T_BLK = 64
UNROLL = 4
B_BLK = 128


def _lstm_kernel(x_ref, wx_ref, wh_ref, b_ref, out_ref, c_ref, h_ref):
    tb = pl.program_id(1)

    @pl.when(tb == 0)
    def _():
        c_ref[...] = jnp.zeros_like(c_ref)
        h_ref[...] = jnp.zeros_like(h_ref)

    wx = wx_ref[...]
    wh = wh_ref[...]
    bias = b_ref[...]

    def body(k, carry_token):
        t0 = k * UNROLL
        c = c_ref[...]
        h = h_ref[...]
        for j in range(UNROLL):
            t = t0 + j
            x_t = x_ref[t]
            g1 = jnp.dot(x_t, wx, preferred_element_type=jnp.float32) + bias
            g2 = jnp.dot(h, wh, preferred_element_type=jnp.float32)
            gates = g1 + g2
            i_g = jax.nn.sigmoid(gates[:, 0:HID])
            f_g = jax.nn.sigmoid(gates[:, HID:2 * HID])
            g_g = jnp.tanh(gates[:, 2 * HID:3 * HID])
            o_g = jax.nn.sigmoid(gates[:, 3 * HID:4 * HID])
            c = f_g * c + i_g * g_g
            h = o_g * jnp.tanh(c)
            out_ref[t] = h
        c_ref[...] = c
        h_ref[...] = h
        return carry_token

    jax.lax.fori_loop(0, T_BLK // UNROLL, body, 0)


def kernel(x, Wx, Wh, b):
    B, T, F = x.shape
    xT = jnp.swapaxes(x, 0, 1)
    b2 = b.reshape(1, 4 * HID)
    grid = (B // B_BLK, T // T_BLK)
    ysT = pl.pallas_call(
        _lstm_kernel,
        out_shape=jax.ShapeDtypeStruct((T, B, HID), x.dtype),
        grid=grid,
        in_specs=[
            pl.BlockSpec((T_BLK, B_BLK, F), lambda bb, tb: (tb, bb, 0)),
            pl.BlockSpec((F, 4 * HID), lambda bb, tb: (0, 0)),
            pl.BlockSpec((HID, 4 * HID), lambda bb, tb: (0, 0)),
            pl.BlockSpec((1, 4 * HID), lambda bb, tb: (0, 0)),
        ],
        out_specs=pl.BlockSpec((T_BLK, B_BLK, HID), lambda bb, tb: (tb, bb, 0)),
        scratch_shapes=[
            pltpu.VMEM((B_BLK, HID), jnp.float32),
            pltpu.VMEM((B_BLK, HID), jnp.float32),
        ],
        compiler_params=pltpu.CompilerParams(
            dimension_semantics=("parallel", "arbitrary"),
        ),
        name="financial_rnn_lstm",
    )(xT, Wx, Wh, b2)
    return jnp.swapaxes(ysT, 0, 1)


# speedup vs baseline: 5.3327x; 2.0271x over previous
import jax
import jax.numpy as jnp
import numpy as np
from jax.experimental import pallas as pl
from jax.experimental.pallas import tpu as pltpu

HID = 32
GRP = 4
FEA = 64
NB = 256
G4 = 4 * HID

_S = np.concatenate([np.ones(64), 2 * np.ones(32), np.ones(32)])
_M = np.concatenate([np.ones(64), 2 * np.ones(32), np.ones(32)])
_D = np.concatenate([np.zeros(64), -np.ones(32), np.zeros(32)])
_CELLCONST = np.stack([_S, _M, _D]).astype(np.float32)


def _lstm_kernel(x_ref, wx4_ref, whp_ref, b4_ref, cc_ref, out_ref, c_ref, h_ref):
    tb = pl.program_id(0)

    @pl.when(tb == 0)
    def _():
        c_ref[...] = jnp.zeros_like(c_ref)
        h_ref[...] = jnp.zeros_like(h_ref)

    cc = cc_ref[...]
    svec = cc[0:1, :]
    mvec = cc[1:2, :]
    dvec = cc[2:3, :]
    whp = whp_ref[...]

    xg4 = jnp.dot(x_ref[...], wx4_ref[...], preferred_element_type=jnp.float32)
    xg4 = xg4 + b4_ref[...]

    c = c_ref[...]
    h = h_ref[...]
    for j in range(GRP):
        gates = xg4[:, j * G4:(j + 1) * G4] + jnp.dot(
            h, whp, preferred_element_type=jnp.float32)
        act = jax.nn.sigmoid(gates * svec) * mvec + dvec
        u = pltpu.roll(act, 3 * HID, 1) * pltpu.roll(act, 2 * HID, 1)
        ro = pltpu.roll(act, HID, 1)
        c = act * c + u
        h = jnp.tanh(c) * ro
        out_ref[:, j * HID:(j + 1) * HID] = h[:, 0:HID]
    c_ref[...] = c
    h_ref[...] = h


def kernel(x, Wx, Wh, b):
    B, T, F = x.shape
    x2 = x.reshape(B, T * F)
    perm = np.concatenate([np.arange(HID, 2 * HID), np.arange(0, HID),
                           np.arange(2 * HID, 4 * HID)])
    wxp = Wx[:, perm]
    whp = jnp.concatenate(
        [Wh[:, perm], jnp.zeros((G4 - HID, G4), Wh.dtype)], axis=0)
    wx4 = jnp.kron(jnp.eye(GRP, dtype=Wx.dtype), wxp)
    b4 = jnp.tile(b[perm], GRP).reshape(1, GRP * G4)
    cellconst = jnp.asarray(_CELLCONST)

    out2 = pl.pallas_call(
        _lstm_kernel,
        out_shape=jax.ShapeDtypeStruct((B, T * HID), x.dtype),
        grid=(T // GRP,),
        in_specs=[
            pl.BlockSpec((NB, GRP * FEA), lambda t: (0, t)),
            pl.BlockSpec((GRP * FEA, GRP * G4), lambda t: (0, 0)),
            pl.BlockSpec((G4, G4), lambda t: (0, 0)),
            pl.BlockSpec((1, GRP * G4), lambda t: (0, 0)),
            pl.BlockSpec((3, G4), lambda t: (0, 0)),
        ],
        out_specs=pl.BlockSpec((NB, GRP * HID), lambda t: (0, t)),
        scratch_shapes=[
            pltpu.VMEM((NB, G4), jnp.float32),
            pltpu.VMEM((NB, G4), jnp.float32),
        ],
        compiler_params=pltpu.CompilerParams(
            dimension_semantics=("arbitrary",),
        ),
        name="financial_rnn_lstm",
    )(x2, wx4, whp, b4, cellconst)
    return out2.reshape(B, T, HID)


# speedup vs baseline: 5.3383x; 1.0010x over previous
import jax
import jax.numpy as jnp
import numpy as np
from jax.experimental import pallas as pl
from jax.experimental.pallas import tpu as pltpu

HID = 32
GRP = 4
FEA = 64
NB = 256
G4 = 4 * HID


def _lstm_kernel(x_ref, wx4_ref, wh3_ref, b4_ref, out_ref, c_ref, h_ref):
    tb = pl.program_id(0)

    @pl.when(tb == 0)
    def _():
        c_ref[...] = jnp.zeros_like(c_ref)
        h_ref[...] = jnp.zeros_like(h_ref)

    wh3 = wh3_ref[...]

    xg4 = jnp.dot(x_ref[...], wx4_ref[...], preferred_element_type=jnp.float32)
    xg4 = xg4 + b4_ref[...]

    c = c_ref[...]
    h = h_ref[...]
    for j in range(GRP):
        xg_j = xg4[:, j * G4:(j + 1) * G4]
        xg1_j = pltpu.roll(xg_j, 3 * HID, 1)
        xg2_j = pltpu.roll(xg_j, 2 * HID, 1)
        hh = jnp.dot(h, wh3, preferred_element_type=jnp.float32)
        g0 = xg_j + hh[:, 0:G4]
        g1 = xg1_j[:, 0:HID] + hh[:, G4:G4 + HID]
        g2 = xg2_j[:, 0:HID] + hh[:, 2 * G4:2 * G4 + HID]
        act0 = jax.nn.sigmoid(g0)
        u = jax.nn.sigmoid(g1) * jnp.tanh(g2)
        c = act0[:, 0:HID] * c + u
        ro = pltpu.roll(act0, HID, 1)
        h = jnp.tanh(c) * ro[:, 0:HID]
        out_ref[:, j * HID:(j + 1) * HID] = h
    c_ref[...] = c
    h_ref[...] = h


def kernel(x, Wx, Wh, b):
    B, T, F = x.shape
    x2 = x.reshape(B, T * F)
    perm = np.concatenate([np.arange(HID, 2 * HID), np.arange(0, HID),
                           np.arange(2 * HID, 4 * HID)])
    s32 = (np.arange(G4) + HID) % G4
    s64 = (np.arange(G4) + 2 * HID) % G4
    wxp = Wx[:, perm]
    whp = Wh[:, perm]
    wh3 = jnp.concatenate([whp, whp[:, s32], whp[:, s64]], axis=1)
    wx4 = jnp.kron(jnp.eye(GRP, dtype=Wx.dtype), wxp)
    b4 = jnp.tile(b[perm], GRP).reshape(1, GRP * G4)

    out2 = pl.pallas_call(
        _lstm_kernel,
        out_shape=jax.ShapeDtypeStruct((B, T * HID), x.dtype),
        grid=(T // GRP,),
        in_specs=[
            pl.BlockSpec((NB, GRP * FEA), lambda t: (0, t)),
            pl.BlockSpec((GRP * FEA, GRP * G4), lambda t: (0, 0)),
            pl.BlockSpec((HID, 3 * G4), lambda t: (0, 0)),
            pl.BlockSpec((1, GRP * G4), lambda t: (0, 0)),
        ],
        out_specs=pl.BlockSpec((NB, GRP * HID), lambda t: (0, t)),
        scratch_shapes=[
            pltpu.VMEM((NB, HID), jnp.float32),
            pltpu.VMEM((NB, HID), jnp.float32),
        ],
        compiler_params=pltpu.CompilerParams(
            dimension_semantics=("arbitrary",),
        ),
        name="financial_rnn_lstm",
    )(x2, wx4, wh3, b4)
    return out2.reshape(B, T, HID)


# speedup vs baseline: 7.0360x; 1.3180x over previous
import jax
import jax.numpy as jnp
import numpy as np
from jax.experimental import pallas as pl
from jax.experimental.pallas import tpu as pltpu

HID = 32
FEA = 64
NB = 256
G4 = 4 * HID
T_BLK = 32
UNROLL = 4


def _lstm_kernel(x_ref, wx_ref, wh3_ref, b_ref, out_ref, c_ref, h_ref):
    tb = pl.program_id(0)

    @pl.when(tb == 0)
    def _():
        c_ref[...] = jnp.zeros_like(c_ref)
        h_ref[...] = jnp.zeros_like(h_ref)

    wx = wx_ref[...]
    wh3 = wh3_ref[...]
    bias = b_ref[...]

    def body(k, carry_token):
        t0 = k * UNROLL
        c = c_ref[...]
        h = h_ref[...]
        for j in range(UNROLL):
            t = t0 + j
            xg = jnp.dot(x_ref[t], wx, preferred_element_type=jnp.float32)
            xg = xg + bias
            xg1 = pltpu.roll(xg, 3 * HID, 1)
            xg2 = pltpu.roll(xg, 2 * HID, 1)
            hh = jnp.dot(h, wh3, preferred_element_type=jnp.float32)
            g0 = xg + hh[:, 0:G4]
            g1 = xg1[:, 0:HID] + hh[:, G4:G4 + HID]
            g2 = xg2[:, 0:HID] + hh[:, 2 * G4:2 * G4 + HID]
            act0 = jax.nn.sigmoid(g0)
            u = jax.nn.sigmoid(g1) * jnp.tanh(g2)
            c = act0[:, 0:HID] * c + u
            ro = pltpu.roll(act0, HID, 1)
            h = jnp.tanh(c) * ro[:, 0:HID]
            out_ref[t] = h
        c_ref[...] = c
        h_ref[...] = h
        return carry_token

    jax.lax.fori_loop(0, T_BLK // UNROLL, body, 0)


def kernel(x, Wx, Wh, b):
    B, T, F = x.shape
    xT = jnp.swapaxes(x, 0, 1)
    perm = np.concatenate([np.arange(HID, 2 * HID), np.arange(0, HID),
                           np.arange(2 * HID, 4 * HID)])
    s32 = (np.arange(G4) + HID) % G4
    s64 = (np.arange(G4) + 2 * HID) % G4
    wxp = Wx[:, perm]
    whp = Wh[:, perm]
    wh3 = jnp.concatenate([whp, whp[:, s32], whp[:, s64]], axis=1)
    b1 = b[perm].reshape(1, G4)

    ysT = pl.pallas_call(
        _lstm_kernel,
        out_shape=jax.ShapeDtypeStruct((T, B, HID), x.dtype),
        grid=(T // T_BLK,),
        in_specs=[
            pl.BlockSpec((T_BLK, NB, FEA), lambda t: (t, 0, 0)),
            pl.BlockSpec((FEA, G4), lambda t: (0, 0)),
            pl.BlockSpec((HID, 3 * G4), lambda t: (0, 0)),
            pl.BlockSpec((1, G4), lambda t: (0, 0)),
        ],
        out_specs=pl.BlockSpec((T_BLK, NB, HID), lambda t: (t, 0, 0)),
        scratch_shapes=[
            pltpu.VMEM((NB, HID), jnp.float32),
            pltpu.VMEM((NB, HID), jnp.float32),
        ],
        compiler_params=pltpu.CompilerParams(
            dimension_semantics=("arbitrary",),
        ),
        name="financial_rnn_lstm",
    )(xT, wxp, wh3, b1)
    return jnp.swapaxes(ysT, 0, 1)


# speedup vs baseline: 7.2035x; 1.0238x over previous
import jax
import jax.numpy as jnp
import numpy as np
from jax.experimental import pallas as pl
from jax.experimental.pallas import tpu as pltpu

HID = 32
FEA = 64
NB = 256
G4 = 4 * HID
T_BLK = 32
UNROLL = 8


def _lstm_kernel(x_ref, wx3_ref, wh3_ref, b3_ref, out_ref, c_ref, h_ref):
    tb = pl.program_id(0)

    @pl.when(tb == 0)
    def _():
        c_ref[...] = jnp.zeros_like(c_ref)
        h_ref[...] = jnp.zeros_like(h_ref)

    wx3 = wx3_ref[...]
    wh3 = wh3_ref[...]
    bias3 = b3_ref[...]

    def body(k, carry_token):
        t0 = k * UNROLL
        c = c_ref[...]
        h = h_ref[...]
        for j in range(UNROLL):
            t = t0 + j
            xgb = jnp.dot(x_ref[t], wx3, preferred_element_type=jnp.float32)
            xgb = xgb + bias3
            hh = jnp.dot(h, wh3, preferred_element_type=jnp.float32)
            a0 = jnp.tanh(xgb[:, 0:G4] + hh[:, 0:G4])
            ai = jnp.tanh(xgb[:, G4:G4 + HID] + hh[:, G4:G4 + HID])
            ag = jnp.tanh(xgb[:, 2 * G4:2 * G4 + HID]
                          + hh[:, 2 * G4:2 * G4 + HID])
            c = (0.5 * a0[:, 0:HID] + 0.5) * c + (0.5 * ai + 0.5) * ag
            ro = pltpu.roll(a0, HID, 1)
            h = (0.5 * ro[:, 0:HID] + 0.5) * jnp.tanh(c)
            out_ref[t] = h
        c_ref[...] = c
        h_ref[...] = h
        return carry_token

    jax.lax.fori_loop(0, T_BLK // UNROLL, body, 0)


def kernel(x, Wx, Wh, b):
    B, T, F = x.shape
    xT = jnp.swapaxes(x, 0, 1)
    perm = np.concatenate([np.arange(HID, 2 * HID), np.arange(0, HID),
                           np.arange(2 * HID, 4 * HID)])
    gscale = np.concatenate([np.full(2 * HID, 0.5), np.ones(HID),
                             np.full(HID, 0.5)]).astype(np.float32)
    s32 = (np.arange(G4) + HID) % G4
    s64 = (np.arange(G4) + 2 * HID) % G4
    wxp = Wx[:, perm] * gscale
    whp = Wh[:, perm] * gscale
    bp = b[perm] * gscale
    wx3 = jnp.concatenate([wxp, wxp[:, s32], wxp[:, s64]], axis=1)
    wh3 = jnp.concatenate([whp, whp[:, s32], whp[:, s64]], axis=1)
    b3 = jnp.concatenate([bp, bp[s32], bp[s64]]).reshape(1, 3 * G4)

    ysT = pl.pallas_call(
        _lstm_kernel,
        out_shape=jax.ShapeDtypeStruct((T, B, HID), x.dtype),
        grid=(T // T_BLK,),
        in_specs=[
            pl.BlockSpec((T_BLK, NB, FEA), lambda t: (t, 0, 0)),
            pl.BlockSpec((FEA, 3 * G4), lambda t: (0, 0)),
            pl.BlockSpec((HID, 3 * G4), lambda t: (0, 0)),
            pl.BlockSpec((1, 3 * G4), lambda t: (0, 0)),
        ],
        out_specs=pl.BlockSpec((T_BLK, NB, HID), lambda t: (t, 0, 0)),
        scratch_shapes=[
            pltpu.VMEM((NB, HID), jnp.float32),
            pltpu.VMEM((NB, HID), jnp.float32),
        ],
        compiler_params=pltpu.CompilerParams(
            dimension_semantics=("arbitrary",),
        ),
        name="financial_rnn_lstm",
    )(xT, wx3, wh3, b3)
    return jnp.swapaxes(ysT, 0, 1)
